# baseline (speedup 1.0000x reference)
import functools

import jax
import jax.numpy as jnp
import numpy as np
from jax import lax
from jax.experimental import pallas as pl
from jax.experimental.pallas import tpu as pltpu
from jax.experimental.pallas import tpu_sc as plsc

EMBED_DIM = 256
N_LEVELS = 4
N_HEADS = 8
N_POINTS = 4
HEAD_DIM = EMBED_DIM // N_HEADS
NQ = 4096
BSZ = 2
HMAX = 64
LP = N_LEVELS * N_POINTS
NCOL = N_HEADS * LP

_SIZES = (64, 32, 16, 8)
_AREAS = tuple(BSZ * s * s for s in _SIZES)
_BASES = (0, 8192, 10240, 10752)
NPOS = 10880
NROWS = NPOS * N_HEADS
R_TOT = NQ * N_HEADS
K_PER_ROW = LP * 4

SC_NC = 2
SC_NS = 16
NW = SC_NC * SC_NS
ROWS_PER_W = R_TOT // NW
CHUNK_ROWS = 16
CHUNK_K = CHUNK_ROWS * K_PER_ROW
N_CHUNKS = ROWS_PER_W // CHUNK_ROWS


def _mm_kernel(x_ref, w_ref, b_ref, o_ref):
    o_ref[...] = (
        jnp.dot(x_ref[...], w_ref[...], preferred_element_type=jnp.float32)
        + b_ref[...]
    )


def _mm(x, w, b, bm=1024):
    m, k = x.shape
    n = w.shape[1]
    return pl.pallas_call(
        _mm_kernel,
        grid=(m // bm,),
        in_specs=[
            pl.BlockSpec((bm, k), lambda i: (i, 0)),
            pl.BlockSpec((k, n), lambda i: (0, 0)),
            pl.BlockSpec((1, n), lambda i: (0, 0)),
        ],
        out_specs=pl.BlockSpec((bm, n), lambda i: (i, 0)),
        out_shape=jax.ShapeDtypeStruct((m, n), jnp.float32),
    )(x, w, b.reshape(1, n))


def _mmtab_kernel(x_ref, w_ref, b_ref, o_ref):
    y = (jnp.dot(x_ref[...], w_ref[...], preferred_element_type=jnp.float32)
         + b_ref[...]).astype(jnp.bfloat16)
    for h in range(N_HEADS):
        o_ref[:, h, :] = y[:, h * HEAD_DIM:(h + 1) * HEAD_DIM]


def _mmtab(x, w, b, bm=1088):
    m, k = x.shape
    n = w.shape[1]
    return pl.pallas_call(
        _mmtab_kernel,
        grid=(m // bm,),
        in_specs=[
            pl.BlockSpec((bm, k), lambda i: (i, 0)),
            pl.BlockSpec((k, n), lambda i: (0, 0)),
            pl.BlockSpec((1, n), lambda i: (0, 0)),
        ],
        out_specs=pl.BlockSpec((bm, N_HEADS, HEAD_DIM), lambda i: (i, 0, 0)),
        out_shape=jax.ShapeDtypeStruct((m, N_HEADS, HEAD_DIM),
                                       jnp.bfloat16),
    )(x, w, b.reshape(1, n))


def _mmheads_kernel(x_ref, w_ref, b_ref, o_ref):
    x = jnp.concatenate([x_ref[:, h, :] for h in range(N_HEADS)], axis=1)
    o_ref[...] = (
        jnp.dot(x, w_ref[...], preferred_element_type=jnp.float32)
        + b_ref[...]
    )


def _mmheads(x, w, b, bm=1024):
    m = x.shape[0]
    n = w.shape[1]
    return pl.pallas_call(
        _mmheads_kernel,
        grid=(m // bm,),
        in_specs=[
            pl.BlockSpec((bm, N_HEADS, HEAD_DIM), lambda i: (i, 0, 0)),
            pl.BlockSpec((EMBED_DIM, n), lambda i: (0, 0)),
            pl.BlockSpec((1, n), lambda i: (0, 0)),
        ],
        out_specs=pl.BlockSpec((bm, n), lambda i: (i, 0)),
        out_shape=jax.ShapeDtypeStruct((m, n), jnp.float32),
    )(x, w, b.reshape(1, n))


def _addr_kernel(q_ref, w_ref, b_ref, rpx_ref, rpy_ref, size_ref, base_ref,
                 head_ref, g_ref, ia_ref, wa_ref, *, bq):
    oa = (jnp.dot(q_ref[...], w_ref[...], preferred_element_type=jnp.float32)
          + b_ref[...])
    offx = oa[:, 0:NCOL]
    offy = oa[:, NCOL:2 * NCOL]
    logits = oa[:, 2 * NCOL:3 * NCOL]
    m = jnp.max(logits, axis=1, keepdims=True)
    e = jnp.exp(logits - m)
    denom = jnp.dot(e, g_ref[...], preferred_element_type=jnp.float32)
    attn = e / denom

    size_f = size_ref[...].astype(jnp.float32)
    lx = rpx_ref[...] + offx / size_f
    ly = rpy_ref[...] + offy / size_f
    lx = lx * 2.0 - 1.0
    ly = ly * 2.0 - 1.0
    x = ((lx + 1.0) * size_f - 1.0) / 2.0
    y = ((ly + 1.0) * size_f - 1.0) / 2.0
    x0 = jnp.floor(x).astype(jnp.int32)
    y0 = jnp.floor(y).astype(jnp.int32)
    x1 = x0 + 1
    y1 = y0 + 1
    size_i = size_ref[...]
    x0 = jnp.clip(x0, 0, size_i - 1)
    x1 = jnp.clip(x1, 0, size_i - 1)
    y0 = jnp.clip(y0, 0, size_i - 1)
    y1 = jnp.clip(y1, 0, size_i - 1)
    x0f = x0.astype(jnp.float32)
    x1f = x1.astype(jnp.float32)
    y0f = y0.astype(jnp.float32)
    y1f = y1.astype(jnp.float32)
    wa = (x1f - x) * (y1f - y) * attn
    wb = (x1f - x) * (y - y0f) * attn
    wc = (x - x0f) * (y1f - y) * attn
    wd = (x - x0f) * (y - y0f) * attn

    pid = pl.program_id(0)
    rows = pid * bq + lax.broadcasted_iota(jnp.int32, (bq, NCOL), 0)
    batch = (rows >= (NQ // BSZ)).astype(jnp.int32)
    base = base_ref[...] + (batch * size_i) * size_i
    hcol = head_ref[...]
    wsh = (bq // 128, 4, 32, NCOL)
    ia_ref[:, :, 0] = ((base + x0 * size_i + y0) * N_HEADS + hcol).reshape(wsh)
    ia_ref[:, :, 1] = ((base + x0 * size_i + y1) * N_HEADS + hcol).reshape(wsh)
    ia_ref[:, :, 2] = ((base + x1 * size_i + y0) * N_HEADS + hcol).reshape(wsh)
    ia_ref[:, :, 3] = ((base + x1 * size_i + y1) * N_HEADS + hcol).reshape(wsh)
    wa_ref[:, :, 0] = wa.reshape(wsh)
    wa_ref[:, :, 1] = wb.reshape(wsh)
    wa_ref[:, :, 2] = wc.reshape(wsh)
    wa_ref[:, :, 3] = wd.reshape(wsh)


def _addr(query, Wcat, bcat, rpx, rpy, size_col, base_col, head_col, G,
          bq=1024):
    n_out = [jax.ShapeDtypeStruct((NW, 4, 4, 32, NCOL), jnp.int32),
             jax.ShapeDtypeStruct((NW, 4, 4, 32, NCOL), jnp.float32)]
    out_spec = pl.BlockSpec((bq // 128, 4, 4, 32, NCOL),
                            lambda i: (i, 0, 0, 0, 0))
    return pl.pallas_call(
        functools.partial(_addr_kernel, bq=bq),
        grid=(NQ // bq,),
        in_specs=[
            pl.BlockSpec((bq, EMBED_DIM), lambda i: (i, 0)),
            pl.BlockSpec((EMBED_DIM, 3 * NCOL), lambda i: (0, 0)),
            pl.BlockSpec((1, 3 * NCOL), lambda i: (0, 0)),
            pl.BlockSpec((bq, 1), lambda i: (i, 0)),
            pl.BlockSpec((bq, 1), lambda i: (i, 0)),
            pl.BlockSpec((1, NCOL), lambda i: (0, 0)),
            pl.BlockSpec((1, NCOL), lambda i: (0, 0)),
            pl.BlockSpec((1, NCOL), lambda i: (0, 0)),
            pl.BlockSpec((NCOL, NCOL), lambda i: (0, 0)),
        ],
        out_specs=[out_spec] * 2,
        out_shape=n_out,
    )(query, Wcat, bcat, rpx, rpy, size_col, base_col, head_col, G)


PHASE_Q = 32
N_PHASES = (NQ // NW) // PHASE_Q
CHUNK_Q = 2
CH_PER_PH = PHASE_Q // CHUNK_Q
CH_ROWS = CHUNK_Q * 4 * 128


def _lane_bcast(wv, j):
    return lax.gather(
        wv, jnp.full((16, 1), j, jnp.int32),
        lax.GatherDimensionNumbers(
            offset_dims=(), collapsed_slice_dims=(0,),
            start_index_map=(0,)),
        slice_sizes=(1,),
        mode=lax.GatherScatterMode.PROMISE_IN_BOUNDS)


def _sc_gather_body(idx_hbm, w_hbm, table_hbm, out_hbm,
                    idx_v, w_v, rows_a, rows_b, out_a, out_b,
                    sem_a, sem_b, sem_out):
    wid = lax.axis_index("c") * SC_NS + lax.axis_index("s")
    rows_bufs = (rows_a, rows_b)
    out_bufs = (out_a, out_b)
    sems = (sem_a, sem_b)

    def fire(j):
        descs = []
        for c in range(4):
            for rr in range(CHUNK_Q):
                src_row = c * PHASE_Q + j * CHUNK_Q + rr
                dst = rows_bufs[j % 2].at[pl.ds((c * CHUNK_Q + rr) * 128,
                                                128)]
                descs.append(pltpu.async_copy(
                    table_hbm.at[idx_v.at[src_row]], dst, sems[j % 2]))
        return descs

    def phase_fn(ph, _):
        q0 = pl.multiple_of((wid * (NQ // NW) + ph * PHASE_Q), PHASE_Q)
        blk = pl.multiple_of((wid * N_PHASES + ph) * 128, 128)
        pltpu.sync_copy(idx_hbm.at[pl.ds(blk, 128)], idx_v)
        pltpu.sync_copy(w_hbm.at[pl.ds(blk * 128, 128 * 128)], w_v)
        descs = fire(0)
        out_descs = [None, None]
        for j in range(CH_PER_PH):
            nxt = fire(j + 1) if j + 1 < CH_PER_PH else []
            for d in descs:
                d.wait()
            descs = nxt
            rows_v = rows_bufs[j % 2]
            out_v = out_bufs[j % 2]
            if out_descs[j % 2] is not None:
                out_descs[j % 2].wait()

            def row_fn(r, _):
                q_loc = r // 8
                h = r % 8
                acc0 = jnp.zeros((16,), jnp.float32)
                acc1 = jnp.zeros((16,), jnp.float32)
                for c in range(4):
                    wv = w_v[pl.ds(
                        (c * PHASE_Q + j * CHUNK_Q + q_loc) * 128 + h * 16,
                        16)]
                    krow = (c * CHUNK_Q + q_loc) * 128 + h * 16
                    for t in range(16):
                        bw = _lane_bcast(wv, t)
                        m = rows_v[krow + t, pl.ds(0, 32)]
                        m0, m1 = plsc.unpack(
                            m, format=plsc.PackFormat.INTERLEAVED)
                        acc0 = acc0 + bw * m0
                        acc1 = acc1 + bw * m1
                out_v[r, pl.ds(0, 16)] = acc0
                out_v[r, pl.ds(16, 16)] = acc1
                return 0

            lax.fori_loop(0, CHUNK_Q * 8, row_fn, 0)
            out_row = pl.multiple_of((q0 + j * CHUNK_Q) * 8, 16)
            out_descs[j % 2] = pltpu.async_copy(
                out_v, out_hbm.at[pl.ds(out_row, CHUNK_Q * 8)], sem_out)
        for d in out_descs:
            if d is not None:
                d.wait()
        return 0

    lax.fori_loop(0, N_PHASES, phase_fn, 0)


def _sc_gather(idx3d, w_flat, table):
    mesh = plsc.VectorSubcoreMesh(core_axis_name="c", subcore_axis_name="s",
                                  num_cores=SC_NC, num_subcores=SC_NS)
    f = pl.kernel(
        _sc_gather_body,
        out_type=jax.ShapeDtypeStruct((R_TOT, HEAD_DIM), jnp.float32),
        mesh=mesh,
        scratch_types=[
            pltpu.VMEM((4 * PHASE_Q, 128), jnp.int32),
            pltpu.VMEM((4 * PHASE_Q * 128,), jnp.float32),
            pltpu.VMEM((CH_ROWS, HEAD_DIM), jnp.bfloat16),
            pltpu.VMEM((CH_ROWS, HEAD_DIM), jnp.bfloat16),
            pltpu.VMEM((CHUNK_Q * 8, HEAD_DIM), jnp.float32),
            pltpu.VMEM((CHUNK_Q * 8, HEAD_DIM), jnp.float32),
            pltpu.SemaphoreType.DMA,
            pltpu.SemaphoreType.DMA,
            pltpu.SemaphoreType.DMA,
        ],
        compiler_params=pltpu.CompilerParams(use_tc_tiling_on_sc=False,
                                             needs_layout_passes=False),
    )
    return f(idx3d, w_flat, table)


def kernel(query, reference_points, value, spatial_shapes, query_offsets,
           W_off, b_off, W_attn, b_attn, W_val, b_val, W_out, b_out):
    Wcat = jnp.concatenate(
        [W_off[:, 0::2], W_off[:, 1::2], W_attn], axis=1)
    bcat = jnp.concatenate(
        [b_off[0::2], b_off[1::2], b_attn], axis=0).reshape(1, 3 * NCOL)
    rpx = reference_points[:, 0:1]
    rpy = reference_points[:, 1:2]
    col = np.arange(NCOL)
    lcol = (col % LP) // N_POINTS
    size_col = jnp.asarray(
        np.array(_SIZES)[lcol].reshape(1, NCOL), jnp.int32)
    base_col = jnp.asarray(
        np.array(_BASES)[lcol].reshape(1, NCOL), jnp.int32)
    head_col = jnp.asarray((col // LP).reshape(1, NCOL), jnp.int32)
    G = jnp.asarray(
        (col.reshape(-1, 1) // LP == col.reshape(1, -1) // LP),
        jnp.float32)

    idx, wts = _addr(
        query, Wcat, bcat, rpx, rpy, size_col, base_col, head_col, G)
    idx_flat = idx.reshape(-1, 128)
    w_flat = wts.reshape(-1)

    vin = jnp.concatenate([
        value[:, :s, :s, l, :].reshape(BSZ * s * s, EMBED_DIM)
        for l, s in enumerate(_SIZES)
    ], axis=0)
    table = _mmtab(vin, W_val, b_val, bm=1088).reshape(NROWS, HEAD_DIM)

    heads = _sc_gather(idx_flat, w_flat, table)

    perm = np.arange(EMBED_DIM).reshape(N_HEADS, 2, HEAD_DIM // 2)
    perm = np.transpose(perm, (0, 2, 1)).reshape(-1)
    inv = np.empty_like(perm)
    inv[perm] = np.arange(EMBED_DIM)
    W_out_p = W_out[jnp.asarray(inv)]
    out = _mmheads(heads.reshape(NQ, N_HEADS, HEAD_DIM), W_out_p, b_out)
    return out


# speedup vs baseline: 1.3609x; 1.3609x over previous
import functools

import jax
import jax.numpy as jnp
import numpy as np
from jax import lax
from jax.experimental import pallas as pl
from jax.experimental.pallas import tpu as pltpu
from jax.experimental.pallas import tpu_sc as plsc

EMBED_DIM = 256
N_LEVELS = 4
N_HEADS = 8
N_POINTS = 4
HEAD_DIM = EMBED_DIM // N_HEADS
NQ = 4096
BSZ = 2
HMAX = 64
LP = N_LEVELS * N_POINTS
NCOL = N_HEADS * LP

_SIZES = (64, 32, 16, 8)
_AREAS = tuple(BSZ * s * s for s in _SIZES)
_BASES = (0, 8192, 10240, 10752)
NPOS = 10880
NROWS = NPOS * N_HEADS
R_TOT = NQ * N_HEADS
K_PER_ROW = LP * 4

SC_NC = 2
SC_NS = 16
NW = SC_NC * SC_NS
ROWS_PER_W = R_TOT // NW
CHUNK_ROWS = 16
CHUNK_K = CHUNK_ROWS * K_PER_ROW
N_CHUNKS = ROWS_PER_W // CHUNK_ROWS


def _mm_kernel(x_ref, w_ref, b_ref, o_ref):
    o_ref[...] = (
        jnp.dot(x_ref[...], w_ref[...], preferred_element_type=jnp.float32)
        + b_ref[...]
    )


def _mm(x, w, b, bm=1024):
    m, k = x.shape
    n = w.shape[1]
    return pl.pallas_call(
        _mm_kernel,
        grid=(m // bm,),
        in_specs=[
            pl.BlockSpec((bm, k), lambda i: (i, 0)),
            pl.BlockSpec((k, n), lambda i: (0, 0)),
            pl.BlockSpec((1, n), lambda i: (0, 0)),
        ],
        out_specs=pl.BlockSpec((bm, n), lambda i: (i, 0)),
        out_shape=jax.ShapeDtypeStruct((m, n), jnp.float32),
    )(x, w, b.reshape(1, n))


def _addr_kernel(q_ref, w_ref, b_ref, rpx_ref, rpy_ref, size_ref, base_ref,
                 head_ref, g_ref, ia_ref, wa_ref, *, bq):
    oa = (jnp.dot(q_ref[...], w_ref[...], preferred_element_type=jnp.float32)
          + b_ref[...])
    offx = oa[:, 0:NCOL]
    offy = oa[:, NCOL:2 * NCOL]
    logits = oa[:, 2 * NCOL:3 * NCOL]
    m = jnp.max(logits, axis=1, keepdims=True)
    e = jnp.exp(logits - m)
    denom = jnp.dot(e, g_ref[...], preferred_element_type=jnp.float32)
    attn = e / denom

    size_f = size_ref[...].astype(jnp.float32)
    lx = rpx_ref[...] + offx / size_f
    ly = rpy_ref[...] + offy / size_f
    lx = lx * 2.0 - 1.0
    ly = ly * 2.0 - 1.0
    x = ((lx + 1.0) * size_f - 1.0) / 2.0
    y = ((ly + 1.0) * size_f - 1.0) / 2.0
    x0 = jnp.floor(x).astype(jnp.int32)
    y0 = jnp.floor(y).astype(jnp.int32)
    x1 = x0 + 1
    y1 = y0 + 1
    size_i = size_ref[...]
    x0 = jnp.clip(x0, 0, size_i - 1)
    x1 = jnp.clip(x1, 0, size_i - 1)
    y0 = jnp.clip(y0, 0, size_i - 1)
    y1 = jnp.clip(y1, 0, size_i - 1)
    x0f = x0.astype(jnp.float32)
    x1f = x1.astype(jnp.float32)
    y0f = y0.astype(jnp.float32)
    y1f = y1.astype(jnp.float32)
    wa = (x1f - x) * (y1f - y) * attn
    wb = (x1f - x) * (y - y0f) * attn
    wc = (x - x0f) * (y1f - y) * attn
    wd = (x - x0f) * (y - y0f) * attn

    pid = pl.program_id(0)
    rows = pid * bq + lax.broadcasted_iota(jnp.int32, (bq, NCOL), 0)
    batch = (rows >= (NQ // BSZ)).astype(jnp.int32)
    base = base_ref[...] + (batch * size_i) * size_i
    hcol = head_ref[...]
    wsh = (bq // 128, 4, 32, NCOL)
    ia_ref[:, :, 0] = ((base + x0 * size_i + y0) * N_HEADS + hcol).reshape(wsh)
    ia_ref[:, :, 1] = ((base + x0 * size_i + y1) * N_HEADS + hcol).reshape(wsh)
    ia_ref[:, :, 2] = ((base + x1 * size_i + y0) * N_HEADS + hcol).reshape(wsh)
    ia_ref[:, :, 3] = ((base + x1 * size_i + y1) * N_HEADS + hcol).reshape(wsh)
    wa_ref[:, :, 0] = wa.reshape(wsh)
    wa_ref[:, :, 1] = wb.reshape(wsh)
    wa_ref[:, :, 2] = wc.reshape(wsh)
    wa_ref[:, :, 3] = wd.reshape(wsh)


def _addr(query, Wcat, bcat, rpx, rpy, size_col, base_col, head_col, G,
          bq=1024):
    n_out = [jax.ShapeDtypeStruct((NW, 4, 4, 32, NCOL), jnp.int32),
             jax.ShapeDtypeStruct((NW, 4, 4, 32, NCOL), jnp.float32)]
    out_spec = pl.BlockSpec((bq // 128, 4, 4, 32, NCOL),
                            lambda i: (i, 0, 0, 0, 0))
    return pl.pallas_call(
        functools.partial(_addr_kernel, bq=bq),
        grid=(NQ // bq,),
        in_specs=[
            pl.BlockSpec((bq, EMBED_DIM), lambda i: (i, 0)),
            pl.BlockSpec((EMBED_DIM, 3 * NCOL), lambda i: (0, 0)),
            pl.BlockSpec((1, 3 * NCOL), lambda i: (0, 0)),
            pl.BlockSpec((bq, 1), lambda i: (i, 0)),
            pl.BlockSpec((bq, 1), lambda i: (i, 0)),
            pl.BlockSpec((1, NCOL), lambda i: (0, 0)),
            pl.BlockSpec((1, NCOL), lambda i: (0, 0)),
            pl.BlockSpec((1, NCOL), lambda i: (0, 0)),
            pl.BlockSpec((NCOL, NCOL), lambda i: (0, 0)),
        ],
        out_specs=[out_spec] * 2,
        out_shape=n_out,
    )(query, Wcat, bcat, rpx, rpy, size_col, base_col, head_col, G)


PHASE_Q = 32
N_PHASES = (NQ // NW) // PHASE_Q
CHUNK_Q = 2
CH_PER_PH = PHASE_Q // CHUNK_Q
CH_ROWS = CHUNK_Q * 4 * 128


def _lane_bcast(wv, j):
    return lax.gather(
        wv, jnp.full((16, 1), j, jnp.int32),
        lax.GatherDimensionNumbers(
            offset_dims=(), collapsed_slice_dims=(0,),
            start_index_map=(0,)),
        slice_sizes=(1,),
        mode=lax.GatherScatterMode.PROMISE_IN_BOUNDS)


def _sc_gather_body(idx_hbm, w_hbm, table_hbm, out_hbm,
                    idx_v, w_v, rows_a, rows_b, out_a, out_b,
                    sem_a, sem_b, sem_out):
    wid = lax.axis_index("c") * SC_NS + lax.axis_index("s")
    rows_bufs = (rows_a, rows_b)
    out_bufs = (out_a, out_b)
    sems = (sem_a, sem_b)

    def fire(j):
        descs = []
        for c in range(4):
            for rr in range(CHUNK_Q):
                src_row = c * PHASE_Q + j * CHUNK_Q + rr
                dst = rows_bufs[j % 2].at[pl.ds((c * CHUNK_Q + rr) * 128,
                                                128)]
                descs.append(pltpu.async_copy(
                    table_hbm.at[idx_v.at[src_row]], dst, sems[j % 2]))
        return descs

    def phase_fn(ph, _):
        q0 = pl.multiple_of((wid * (NQ // NW) + ph * PHASE_Q), PHASE_Q)
        blk = pl.multiple_of((wid * N_PHASES + ph) * 128, 128)
        pltpu.sync_copy(idx_hbm.at[pl.ds(blk, 128)], idx_v)
        pltpu.sync_copy(w_hbm.at[pl.ds(blk * 128, 128 * 128)], w_v)
        descs = fire(0)
        out_descs = [None, None]
        for j in range(CH_PER_PH):
            nxt = fire(j + 1) if j + 1 < CH_PER_PH else []
            for d in descs:
                d.wait()
            descs = nxt
            rows_v = rows_bufs[j % 2]
            out_v = out_bufs[j % 2]
            if out_descs[j % 2] is not None:
                out_descs[j % 2].wait()

            def row_fn(r, _):
                q_loc = r // 8
                h = r % 8
                acc0 = jnp.zeros((16,), jnp.float32)
                acc1 = jnp.zeros((16,), jnp.float32)
                for c in range(4):
                    wv = w_v[pl.ds(
                        (c * PHASE_Q + j * CHUNK_Q + q_loc) * 128 + h * 16,
                        16)]
                    krow = (c * CHUNK_Q + q_loc) * 128 + h * 16
                    for t in range(16):
                        bw = _lane_bcast(wv, t)
                        m = rows_v[krow + t, pl.ds(0, 32)]
                        m0, m1 = plsc.unpack(
                            m, format=plsc.PackFormat.INTERLEAVED)
                        acc0 = acc0 + bw * m0
                        acc1 = acc1 + bw * m1
                out_v[r, pl.ds(0, 16)] = acc0
                out_v[r, pl.ds(16, 16)] = acc1
                return 0

            lax.fori_loop(0, CHUNK_Q * 8, row_fn, 0)
            out_row = pl.multiple_of((q0 + j * CHUNK_Q) * 8, 16)
            out_descs[j % 2] = pltpu.async_copy(
                out_v, out_hbm.at[pl.ds(out_row, CHUNK_Q * 8)], sem_out)
        for d in out_descs:
            if d is not None:
                d.wait()
        return 0

    lax.fori_loop(0, N_PHASES, phase_fn, 0)


def _sc_gather(idx3d, w_flat, table):
    mesh = plsc.VectorSubcoreMesh(core_axis_name="c", subcore_axis_name="s",
                                  num_cores=SC_NC, num_subcores=SC_NS)
    f = pl.kernel(
        _sc_gather_body,
        out_type=jax.ShapeDtypeStruct((R_TOT, HEAD_DIM), jnp.float32),
        mesh=mesh,
        scratch_types=[
            pltpu.VMEM((4 * PHASE_Q, 128), jnp.int32),
            pltpu.VMEM((4 * PHASE_Q * 128,), jnp.float32),
            pltpu.VMEM((CH_ROWS, HEAD_DIM), jnp.bfloat16),
            pltpu.VMEM((CH_ROWS, HEAD_DIM), jnp.bfloat16),
            pltpu.VMEM((CHUNK_Q * 8, HEAD_DIM), jnp.float32),
            pltpu.VMEM((CHUNK_Q * 8, HEAD_DIM), jnp.float32),
            pltpu.SemaphoreType.DMA,
            pltpu.SemaphoreType.DMA,
            pltpu.SemaphoreType.DMA,
        ],
        compiler_params=pltpu.CompilerParams(use_tc_tiling_on_sc=False,
                                             needs_layout_passes=False),
    )
    return f(idx3d, w_flat, table)


def kernel(query, reference_points, value, spatial_shapes, query_offsets,
           W_off, b_off, W_attn, b_attn, W_val, b_val, W_out, b_out):
    Wcat = jnp.concatenate(
        [W_off[:, 0::2], W_off[:, 1::2], W_attn], axis=1)
    bcat = jnp.concatenate(
        [b_off[0::2], b_off[1::2], b_attn], axis=0).reshape(1, 3 * NCOL)
    rpx = reference_points[:, 0:1]
    rpy = reference_points[:, 1:2]
    col = np.arange(NCOL)
    lcol = (col % LP) // N_POINTS
    size_col = jnp.asarray(
        np.array(_SIZES)[lcol].reshape(1, NCOL), jnp.int32)
    base_col = jnp.asarray(
        np.array(_BASES)[lcol].reshape(1, NCOL), jnp.int32)
    head_col = jnp.asarray((col // LP).reshape(1, NCOL), jnp.int32)
    G = jnp.asarray(
        (col.reshape(-1, 1) // LP == col.reshape(1, -1) // LP),
        jnp.float32)

    idx, wts = _addr(
        query, Wcat, bcat, rpx, rpy, size_col, base_col, head_col, G)
    idx_flat = idx.reshape(-1, 128)
    w_flat = wts.reshape(-1)

    vin = jnp.concatenate([
        value[:, :s, :s, l, :].reshape(BSZ * s * s, EMBED_DIM)
        for l, s in enumerate(_SIZES)
    ], axis=0)
    table = (_mm(vin, W_val, b_val, bm=1088)
             .astype(jnp.bfloat16).reshape(NROWS, HEAD_DIM))

    heads = _sc_gather(idx_flat, w_flat, table)

    perm = np.arange(EMBED_DIM).reshape(N_HEADS, 2, HEAD_DIM // 2)
    perm = np.transpose(perm, (0, 2, 1)).reshape(-1)
    inv = np.empty_like(perm)
    inv[perm] = np.arange(EMBED_DIM)
    W_out_p = W_out[jnp.asarray(inv)]
    out = _mm(heads.reshape(NQ, EMBED_DIM), W_out_p, b_out)
    return out
